# baseline (speedup 1.0000x reference)
import jax
import jax.numpy as jnp
from jax import lax
from jax.experimental import pallas as pl
from jax.experimental.pallas import tpu as pltpu
from jax.experimental.pallas import tpu_sc as plsc

_NUM_PARTICLES = 500000
_D = 64
_N = _NUM_PARTICLES * _D
_NC = 2
_NS = 16
_NW = _NC * _NS
_PER_W = _N // _NW
_CHUNK = 50_000
_NCH = _PER_W // _CHUNK


def _copy_body(src_hbm, out_hbm, buf0, buf1, isem0, isem1, osem0, osem1):
    wid = lax.axis_index("s") * _NC + lax.axis_index("c")
    base = wid * _PER_W
    bufs = (buf0, buf1)
    isems = (isem0, isem1)
    osems = (osem0, osem1)

    def in_cp(i):
        b = i % 2
        return pltpu.make_async_copy(
            src_hbm.at[pl.ds(base + i * _CHUNK, _CHUNK)], bufs[b], isems[b])

    def out_cp(i):
        b = i % 2
        return pltpu.make_async_copy(
            bufs[b], out_hbm.at[pl.ds(base + i * _CHUNK, _CHUNK)], osems[b])

    in_cp(0).start()
    for i in range(_NCH):
        if i >= 1:
            out_cp(i - 1).wait()
        if i + 1 < _NCH:
            in_cp(i + 1).start()
        in_cp(i).wait()
        out_cp(i).start()
    out_cp(_NCH - 1).wait()


@jax.jit
def kernel(sample, particles):
    del sample
    flat = particles.reshape(_N)
    mesh = plsc.VectorSubcoreMesh(core_axis_name="c", subcore_axis_name="s")
    out = pl.kernel(
        _copy_body,
        out_type=jax.ShapeDtypeStruct((_N,), jnp.float32),
        mesh=mesh,
        scratch_types=[
            pltpu.VMEM((_CHUNK,), jnp.float32),
            pltpu.VMEM((_CHUNK,), jnp.float32),
            pltpu.SemaphoreType.DMA,
            pltpu.SemaphoreType.DMA,
            pltpu.SemaphoreType.DMA,
            pltpu.SemaphoreType.DMA,
        ],
    )(flat)
    return out.reshape(_NUM_PARTICLES, _D)
